# baseline (speedup 1.0000x reference)
import functools

import numpy as np
import jax
import jax.numpy as jnp
from jax import lax
from jax.experimental import pallas as pl
from jax.experimental.pallas import tpu as pltpu
from jax.experimental.pallas import tpu_sc as plsc

EMBED_DIM = 128
N_NODES = 10000
N_EDGES = 320000
NUM_RELS = 100
BATCH = 1024
NUM_H = 4

HW = 64
EP2 = 2 * HW
TROWS = 10112
ACC_ROWS = 10240
X2_COLS = NUM_H * EMBED_DIM

NC, NS = 2, 16
NW = NC * NS
EB = 64
E_PAD = 327680
EPW = E_PAD // NW
NBLK = EPW // EB
RTAB_ROWS = 112
ZB = 8


def _dft_mats():
    n = EMBED_DIM
    j = np.arange(n)
    ki = np.arange(n // 2 + 1)
    ang = 2.0 * np.pi / n * np.outer(j, ki)
    fc, fs = np.cos(ang), np.sin(ang)
    w = np.where((ki == 0) | (2 * ki == n), 1.0, 2.0)
    iang = 2.0 * np.pi / n * np.outer(ki, j)
    gr = (w[:, None] * np.cos(iang)) / n
    gi = (w[:, None] * np.sin(iang)) / n
    f2 = np.zeros((n, n), np.float32)
    f2[:, :HW] = fc[:, :HW]
    f2[:, HW] = fc[:, HW]
    f2[:, HW + 1:] = -fs[:, 1:HW]
    g2 = np.zeros((n, n), np.float32)
    g2[:HW] = gr[:HW]
    g2[HW] = gr[HW]
    g2[HW + 1:] = -gi[1:HW]
    return f2, g2


_F2P, _G2P = _dft_mats()


def _spectrum_body(x_ref, f_ref, o_ref):
    o_ref[...] = jnp.dot(x_ref[...], f_ref[...],
                         preferred_element_type=jnp.float32)


def _kmat_body(g_ref, w_ref, o_ref):
    o_ref[0] = jnp.dot(g_ref[...], w_ref[0],
                       preferred_element_type=jnp.float32)


def _mid_body(a_ref, k_ref, b_ref, f_ref, o_ref):
    agg = a_ref[0] + a_ref[1]
    z = jnp.maximum(
        jnp.dot(agg, k_ref[0], preferred_element_type=jnp.float32)
        + b_ref[0], 0.0)
    o_ref[0] = jnp.dot(z, f_ref[...], preferred_element_type=jnp.float32)


def _out_body(a_ref, k_ref, b_ref, bidx_ref, o_ref):
    m = pl.program_id(1)
    agg = a_ref[0, 0] + a_ref[0, 1]
    x = jnp.maximum(
        jnp.dot(agg, k_ref[0], preferred_element_type=jnp.float32)
        + b_ref[0], 0.0)
    bb = bidx_ref[0, 0]
    oh = (bb[:, None]
          == lax.broadcasted_iota(jnp.int32, (128, BATCH), 1)
          ).astype(jnp.float32)
    y = jax.lax.dot_general(oh, x, (((0,), (0,)), ((), ())),
                            preferred_element_type=jnp.float32)

    @pl.when(m == 0)
    def _():
        o_ref[...] = jnp.zeros_like(o_ref)
    o_ref[...] += y


def _tc_spectrum(x):
    m = x.shape[0]
    return pl.pallas_call(
        _spectrum_body,
        grid=(m // 128,),
        in_specs=[pl.BlockSpec((128, EMBED_DIM), lambda i: (i, 0)),
                  pl.BlockSpec((EMBED_DIM, EP2), lambda i: (0, 0))],
        out_specs=pl.BlockSpec((128, EP2), lambda i: (i, 0)),
        out_shape=jax.ShapeDtypeStruct((m, EP2), jnp.float32),
    )(x, jnp.asarray(_F2P))


def _tc_kmats(wf):
    return pl.pallas_call(
        _kmat_body,
        grid=(8,),
        in_specs=[pl.BlockSpec((EP2, EMBED_DIM), lambda i: (0, 0)),
                  pl.BlockSpec((1, EMBED_DIM, EMBED_DIM), lambda i: (i, 0, 0))],
        out_specs=pl.BlockSpec((1, EP2, EMBED_DIM), lambda i: (i, 0, 0)),
        out_shape=jax.ShapeDtypeStruct((8, EP2, EMBED_DIM), jnp.float32),
    )(jnp.asarray(_G2P), wf)


def _tc_layer_mid(acc1, kmats, bf):
    return pl.pallas_call(
        _mid_body,
        grid=(NUM_H, TROWS // 128),
        in_specs=[
            pl.BlockSpec((2, 128, EP2), lambda h, m: (0, m, 0)),
            pl.BlockSpec((1, EP2, EMBED_DIM), lambda h, m: (2 * h, 0, 0)),
            pl.BlockSpec((1, 1, EMBED_DIM), lambda h, m: (2 * h, 0, 0)),
            pl.BlockSpec((EMBED_DIM, EP2), lambda h, m: (0, 0)),
        ],
        out_specs=pl.BlockSpec((1, 128, EP2), lambda h, m: (h, m, 0)),
        out_shape=jax.ShapeDtypeStruct((NUM_H, TROWS, EP2), jnp.float32),
    )(acc1, kmats, bf, jnp.asarray(_F2P))


def _tc_layer_out(acc2s, kmats, bf, bidx3):
    return pl.pallas_call(
        _out_body,
        grid=(NUM_H, ACC_ROWS // 128),
        in_specs=[
            pl.BlockSpec((1, 2, 128, EP2), lambda h, m: (h, 0, m, 0)),
            pl.BlockSpec((1, EP2, EMBED_DIM), lambda h, m: (2 * h + 1, 0, 0)),
            pl.BlockSpec((1, 1, EMBED_DIM), lambda h, m: (2 * h + 1, 0, 0)),
            pl.BlockSpec((1, 1, 128), lambda h, m: (m, 0, 0)),
        ],
        out_specs=pl.BlockSpec((BATCH, EMBED_DIM), lambda h, m: (0, h)),
        out_shape=jax.ShapeDtypeStruct((BATCH, X2_COLS), jnp.float32),
    )(acc2s, kmats, bf, bidx3)


def _compute_block(xrows, rrows, m0):
    def pe(e, cc):
        for q in range(HW // 16):
            xa = xrows[e, pl.ds(q * 16, 16)]
            xb = xrows[e, pl.ds(HW + q * 16, 16)]
            ra = rrows[e, pl.ds(q * 16, 16)]
            rb = rrows[e, pl.ds(HW + q * 16, 16)]
            t1 = xa * ra
            t2 = xb * rb
            pre = t1 + t2
            pim = xa * rb - xb * ra
            if q == 0:
                pre = jnp.where(m0, t1, pre)
                pim = jnp.where(m0, t2, pim)
            xrows[e, pl.ds(q * 16, 16)] = pre
            xrows[e, pl.ds(HW + q * 16, 16)] = pim
        return cc
    lax.fori_loop(0, EB, pe, 0)


def _edge_body(tab, rtabh, sdt, out,
               i0, i1, d0, d1, x0, x1, r0, r1, zbuf,
               acc, isa, isb, gsa, gsb, ssa, ssb):
    c = lax.axis_index("c")
    s = lax.axis_index("s")
    wid = s * NC + c
    z16 = jnp.zeros((16,), jnp.float32)
    m0 = lax.broadcasted_iota(jnp.int32, (16,), 0) == 0

    idx = (i0, i1)
    didx = (d0, d1)
    xb = (x0, x1)
    rb = (r0, r1)
    isx = (isa, isb)
    gs = (gsa, gsb)
    ss = (ssa, ssb)

    def zb(i, carry):
        for q in range(EP2 // 16):
            zbuf[i, pl.ds(q * 16, 16)] = z16
        return carry
    lax.fori_loop(0, ZB, zb, 0)

    rows_per_sub = ACC_ROWS // NS

    def za(i, carry):
        pltpu.sync_copy(zbuf, acc.at[pl.ds(s * rows_per_sub + i * ZB, ZB)])
        return carry
    lax.fori_loop(0, rows_per_sub // ZB, za, 0)
    plsc.subcore_barrier()

    bbase = wid * NBLK

    def idx_issue(k, u):
        pltpu.async_copy(sdt.at[bbase + k], idx[u], isx[u])

    def idx_wait(u):
        pltpu.make_async_copy(sdt.at[bbase], idx[u], isx[u]).wait()

    def gather_issue(u):
        pltpu.async_copy(tab.at[idx[u].at[0]], xb[u], gs[u])
        pltpu.async_copy(rtabh.at[idx[u].at[1]], rb[u], gs[u])

    def gather_wait(u):
        pltpu.make_async_copy(tab.at[idx[u].at[0]], xb[u], gs[u]).wait()
        pltpu.make_async_copy(rtabh.at[idx[u].at[1]], rb[u], gs[u]).wait()

    def scat_issue(u):
        pltpu.async_copy(xb[u], acc.at[didx[u]], ss[u], add=True)

    def scat_wait(u):
        pltpu.make_async_copy(xb[u], acc.at[didx[u]], ss[u]).wait()

    def save_didx(u):
        for j in range(EB // 16):
            didx[u][pl.ds(j * 16, 16)] = idx[u][2, pl.ds(j * 16, 16)]

    def step(k, cur, first):
        nxt = 1 - cur
        gather_wait(cur)
        if not first:
            scat_wait(nxt)
        idx_wait(nxt)
        gather_issue(nxt)
        save_didx(cur)

        @pl.when(k <= NBLK - 3)
        def _():
            idx_issue(k + 2, cur)
        _compute_block(xb[cur], rb[cur], m0)
        scat_issue(cur)

    idx_issue(0, 0)
    idx_wait(0)
    gather_issue(0)
    idx_issue(1, 1)
    step(jnp.int32(0), 0, True)
    step(jnp.int32(1), 1, False)

    def pair(i, carry):
        step(2 * i, 0, False)
        step(2 * i + 1, 1, False)
        return carry
    lax.fori_loop(1, NBLK // 2 - 1, pair, 0)
    step(jnp.int32(NBLK - 2), 0, False)
    gather_wait(1)
    scat_wait(0)
    save_didx(1)
    _compute_block(xb[1], rb[1], m0)
    scat_issue(1)
    scat_wait(1)
    plsc.subcore_barrier()

    def co(i, carry):
        r0_ = s * rows_per_sub + i * 64
        pltpu.sync_copy(acc.at[pl.ds(r0_, 64)], out.at[c, pl.ds(r0_, 64)])
        return carry
    lax.fori_loop(0, rows_per_sub // 64, co, 0)


@functools.lru_cache(maxsize=None)
def _build_sc_kernels():
    mesh = plsc.VectorSubcoreMesh(core_axis_name="c", subcore_axis_name="s")
    edge = pl.kernel(
        _edge_body, mesh=mesh,
        out_type=jax.ShapeDtypeStruct((NC, ACC_ROWS, EP2), jnp.float32),
        scratch_types=[
            pltpu.VMEM((3, EB), jnp.int32),
            pltpu.VMEM((3, EB), jnp.int32),
            pltpu.VMEM((EB,), jnp.int32),
            pltpu.VMEM((EB,), jnp.int32),
            pltpu.VMEM((EB, EP2), jnp.float32),
            pltpu.VMEM((EB, EP2), jnp.float32),
            pltpu.VMEM((EB, EP2), jnp.float32),
            pltpu.VMEM((EB, EP2), jnp.float32),
            pltpu.VMEM((ZB, EP2), jnp.float32),
            pltpu.VMEM_SHARED((ACC_ROWS, EP2), jnp.float32),
            pltpu.SemaphoreType.DMA,
            pltpu.SemaphoreType.DMA,
            pltpu.SemaphoreType.DMA,
            pltpu.SemaphoreType.DMA,
            pltpu.SemaphoreType.DMA,
            pltpu.SemaphoreType.DMA,
        ],
    )
    return edge


def kernel(ent_e, edge_index, edge_type, batch_idx, rel_table, W, b):
    edge_pass = _build_sc_kernels()

    npad = E_PAD - N_EDGES
    src = jnp.concatenate([edge_index[0], jnp.zeros((npad,), jnp.int32)])
    dst = jnp.concatenate(
        [edge_index[1],
         TROWS + (jnp.arange(npad, dtype=jnp.int32) % 128)])
    typ = jnp.concatenate([edge_type, jnp.zeros((npad,), jnp.int32)])
    sdt = jnp.stack([src.reshape(-1, EB), typ.reshape(-1, EB),
                     dst.reshape(-1, EB)], axis=1)

    tin = jnp.concatenate(
        [ent_e, rel_table,
         jnp.zeros((TROWS - N_NODES - NUM_RELS, EMBED_DIM), jnp.float32)],
        axis=0)
    t1 = _tc_spectrum(tin)
    kmats = _tc_kmats(W.reshape(8, EMBED_DIM, EMBED_DIM))
    bf = b.reshape(8, 1, EMBED_DIM)

    rtabh = t1[N_NODES:N_NODES + RTAB_ROWS]
    acc1 = edge_pass(t1, rtabh, sdt)
    mid = _tc_layer_mid(acc1[:, :TROWS], kmats, bf)

    acc2 = [edge_pass(mid[h], rtabh, sdt) for h in range(NUM_H)]
    acc2s = jnp.stack(acc2, axis=0)

    bidx3 = jnp.concatenate(
        [batch_idx,
         jnp.full((ACC_ROWS - N_NODES,), BATCH, jnp.int32)]
    ).reshape(ACC_ROWS // 128, 1, 128)
    return _tc_layer_out(acc2s, kmats, bf, bidx3)


# speedup vs baseline: 1.1473x; 1.1473x over previous
import functools

import numpy as np
import jax
import jax.numpy as jnp
from jax import lax
from jax.experimental import pallas as pl
from jax.experimental.pallas import tpu as pltpu
from jax.experimental.pallas import tpu_sc as plsc

EMBED_DIM = 128
N_NODES = 10000
N_EDGES = 320000
NUM_RELS = 100
BATCH = 1024
NUM_H = 4

HW = 64
EP2 = 2 * HW
TROWS = 10112
ACC_ROWS = 10240
X2_COLS = NUM_H * EMBED_DIM

NC, NS = 2, 16
NW = NC * NS
EB = 64
E_PAD = 327680
EPW = E_PAD // NW
NBLK = EPW // EB
RTAB_ROWS = 112
ZB = 8


def _dft_mats():
    n = EMBED_DIM
    j = np.arange(n)
    ki = np.arange(n // 2 + 1)
    ang = 2.0 * np.pi / n * np.outer(j, ki)
    fc, fs = np.cos(ang), np.sin(ang)
    w = np.where((ki == 0) | (2 * ki == n), 1.0, 2.0)
    iang = 2.0 * np.pi / n * np.outer(ki, j)
    gr = (w[:, None] * np.cos(iang)) / n
    gi = (w[:, None] * np.sin(iang)) / n
    f2 = np.zeros((n, n), np.float32)
    f2[:, :HW] = fc[:, :HW]
    f2[:, HW] = fc[:, HW]
    f2[:, HW + 1:] = -fs[:, 1:HW]
    g2 = np.zeros((n, n), np.float32)
    g2[:HW] = gr[:HW]
    g2[HW] = gr[HW]
    g2[HW + 1:] = -gi[1:HW]
    return f2, g2


_F2P, _G2P = _dft_mats()


def _spectrum_body(x_ref, f_ref, o_ref):
    o_ref[...] = jnp.dot(x_ref[...], f_ref[...],
                         preferred_element_type=jnp.float32)


def _kmat_body(g_ref, w_ref, o_ref):
    o_ref[0] = jnp.dot(g_ref[...], w_ref[0],
                       preferred_element_type=jnp.float32)


def _mid_body(a_ref, k_ref, b_ref, f_ref, o_ref):
    agg = a_ref[0] + a_ref[1]
    z = jnp.maximum(
        jnp.dot(agg, k_ref[0], preferred_element_type=jnp.float32)
        + b_ref[0], 0.0)
    o_ref[0] = jnp.dot(z, f_ref[...], preferred_element_type=jnp.float32)


def _out_body(a_ref, k_ref, b_ref, bidx_ref, o_ref):
    m = pl.program_id(1)
    agg = a_ref[0, 0] + a_ref[0, 1]
    x = jnp.maximum(
        jnp.dot(agg, k_ref[0], preferred_element_type=jnp.float32)
        + b_ref[0], 0.0)
    bb = bidx_ref[0, 0]
    oh = (bb[:, None]
          == lax.broadcasted_iota(jnp.int32, (128, BATCH), 1)
          ).astype(jnp.float32)
    y = jax.lax.dot_general(oh, x, (((0,), (0,)), ((), ())),
                            preferred_element_type=jnp.float32)

    @pl.when(m == 0)
    def _():
        o_ref[...] = jnp.zeros_like(o_ref)
    o_ref[...] += y


def _tc_spectrum(x):
    m = x.shape[0]
    return pl.pallas_call(
        _spectrum_body,
        grid=(m // 128,),
        in_specs=[pl.BlockSpec((128, EMBED_DIM), lambda i: (i, 0)),
                  pl.BlockSpec((EMBED_DIM, EP2), lambda i: (0, 0))],
        out_specs=pl.BlockSpec((128, EP2), lambda i: (i, 0)),
        out_shape=jax.ShapeDtypeStruct((m, EP2), jnp.float32),
    )(x, jnp.asarray(_F2P))


def _tc_kmats(wf):
    return pl.pallas_call(
        _kmat_body,
        grid=(8,),
        in_specs=[pl.BlockSpec((EP2, EMBED_DIM), lambda i: (0, 0)),
                  pl.BlockSpec((1, EMBED_DIM, EMBED_DIM), lambda i: (i, 0, 0))],
        out_specs=pl.BlockSpec((1, EP2, EMBED_DIM), lambda i: (i, 0, 0)),
        out_shape=jax.ShapeDtypeStruct((8, EP2, EMBED_DIM), jnp.float32),
    )(jnp.asarray(_G2P), wf)


def _tc_layer_mid(acc1, kmats, bf):
    return pl.pallas_call(
        _mid_body,
        grid=(NUM_H, TROWS // 128),
        in_specs=[
            pl.BlockSpec((2, 128, EP2), lambda h, m: (0, m, 0)),
            pl.BlockSpec((1, EP2, EMBED_DIM), lambda h, m: (2 * h, 0, 0)),
            pl.BlockSpec((1, 1, EMBED_DIM), lambda h, m: (2 * h, 0, 0)),
            pl.BlockSpec((EMBED_DIM, EP2), lambda h, m: (0, 0)),
        ],
        out_specs=pl.BlockSpec((1, 128, EP2), lambda h, m: (h, m, 0)),
        out_shape=jax.ShapeDtypeStruct((NUM_H, TROWS, EP2), jnp.float32),
    )(acc1, kmats, bf, jnp.asarray(_F2P))


def _tc_layer_out(acc2s, kmats, bf, bidx3):
    return pl.pallas_call(
        _out_body,
        grid=(NUM_H, ACC_ROWS // 128),
        in_specs=[
            pl.BlockSpec((1, 2, 128, EP2), lambda h, m: (h, 0, m, 0)),
            pl.BlockSpec((1, EP2, EMBED_DIM), lambda h, m: (2 * h + 1, 0, 0)),
            pl.BlockSpec((1, 1, EMBED_DIM), lambda h, m: (2 * h + 1, 0, 0)),
            pl.BlockSpec((1, 1, 128), lambda h, m: (m, 0, 0)),
        ],
        out_specs=pl.BlockSpec((BATCH, EMBED_DIM), lambda h, m: (0, h)),
        out_shape=jax.ShapeDtypeStruct((BATCH, X2_COLS), jnp.float32),
    )(acc2s, kmats, bf, bidx3)


def _compute_block(xrows, rrows, m0):
    def pe(e, cc):
        for q in range(HW // 16):
            xa = xrows[e, pl.ds(q * 16, 16)]
            xb = xrows[e, pl.ds(HW + q * 16, 16)]
            ra = rrows[e, pl.ds(q * 16, 16)]
            rb = rrows[e, pl.ds(HW + q * 16, 16)]
            t1 = xa * ra
            t2 = xb * rb
            pre = t1 + t2
            pim = xa * rb - xb * ra
            if q == 0:
                pre = jnp.where(m0, t1, pre)
                pim = jnp.where(m0, t2, pim)
            xrows[e, pl.ds(q * 16, 16)] = pre
            xrows[e, pl.ds(HW + q * 16, 16)] = pim
        return cc
    lax.fori_loop(0, EB, pe, 0)


def _edge_body(tab, rtabh, sdt, out,
               i0, i1, d0, d1, x0, x1, r0, r1, zbuf,
               acc, isa, isb, gsa, gsb, ssa, ssb):
    c = lax.axis_index("c")
    s = lax.axis_index("s")
    wid = s * NC + c
    z16 = jnp.zeros((16,), jnp.float32)
    m0 = lax.broadcasted_iota(jnp.int32, (16,), 0) == 0

    idx = (i0, i1)
    didx = (d0, d1)
    xb = (x0, x1)
    rb = (r0, r1)
    isx = (isa, isb)
    gs = (gsa, gsb)
    ss = (ssa, ssb)

    def zb(i, carry):
        for q in range(EP2 // 16):
            zbuf[i, pl.ds(q * 16, 16)] = z16
        return carry
    lax.fori_loop(0, ZB, zb, 0)

    rows_per_sub = ACC_ROWS // NS

    def za(i, carry):
        pltpu.sync_copy(zbuf, acc.at[pl.ds(s * rows_per_sub + i * ZB, ZB)])
        return carry
    lax.fori_loop(0, rows_per_sub // ZB, za, 0)
    plsc.subcore_barrier()

    bbase = wid * NBLK

    def idx_issue(k, u):
        pltpu.async_copy(sdt.at[bbase + k], idx[u], isx[u])

    def idx_wait(u):
        pltpu.make_async_copy(sdt.at[bbase], idx[u], isx[u]).wait()

    def gather_issue(u):
        pltpu.async_copy(tab.at[idx[u].at[0]], xb[u], gs[u])
        pltpu.async_copy(rtabh.at[idx[u].at[1]], rb[u], gs[u])

    def gather_wait(u):
        pltpu.make_async_copy(tab.at[idx[u].at[0]], xb[u], gs[u]).wait()
        pltpu.make_async_copy(rtabh.at[idx[u].at[1]], rb[u], gs[u]).wait()

    def scat_issue(u):
        pltpu.async_copy(xb[u], acc.at[didx[u]], ss[u], add=True)

    def scat_wait(u):
        pltpu.make_async_copy(xb[u], acc.at[didx[u]], ss[u]).wait()

    def save_didx(u):
        for j in range(EB // 16):
            didx[u][pl.ds(j * 16, 16)] = idx[u][2, pl.ds(j * 16, 16)]

    def step(k, cur, first):
        nxt = 1 - cur
        gather_wait(cur)
        if not first:
            scat_wait(nxt)
        idx_wait(nxt)
        gather_issue(nxt)
        save_didx(cur)

        @pl.when(k <= NBLK - 3)
        def _():
            idx_issue(k + 2, cur)
        _compute_block(xb[cur], rb[cur], m0)
        scat_issue(cur)

    idx_issue(0, 0)
    idx_wait(0)
    gather_issue(0)
    idx_issue(1, 1)
    step(jnp.int32(0), 0, True)
    step(jnp.int32(1), 1, False)

    def pair(i, carry):
        step(2 * i, 0, False)
        step(2 * i + 1, 1, False)
        return carry
    lax.fori_loop(1, NBLK // 2 - 1, pair, 0)
    step(jnp.int32(NBLK - 2), 0, False)
    gather_wait(1)
    scat_wait(0)
    save_didx(1)
    _compute_block(xb[1], rb[1], m0)
    scat_issue(1)
    scat_wait(1)
    plsc.subcore_barrier()

    def co(i, carry):
        r0_ = s * rows_per_sub + i * 64
        pltpu.sync_copy(acc.at[pl.ds(r0_, 64)], out.at[c, pl.ds(r0_, 64)])
        return carry
    lax.fori_loop(0, rows_per_sub // 64, co, 0)


@functools.lru_cache(maxsize=None)
def _build_sc_kernels():
    mesh = plsc.VectorSubcoreMesh(core_axis_name="c", subcore_axis_name="s")
    edge = pl.kernel(
        _edge_body, mesh=mesh,
        out_type=jax.ShapeDtypeStruct((NC, ACC_ROWS, EP2), jnp.float32),
        scratch_types=[
            pltpu.VMEM((3, EB), jnp.int32),
            pltpu.VMEM((3, EB), jnp.int32),
            pltpu.VMEM((EB,), jnp.int32),
            pltpu.VMEM((EB,), jnp.int32),
            pltpu.VMEM((EB, EP2), jnp.float32),
            pltpu.VMEM((EB, EP2), jnp.float32),
            pltpu.VMEM((EB, EP2), jnp.float32),
            pltpu.VMEM((EB, EP2), jnp.float32),
            pltpu.VMEM((ZB, EP2), jnp.float32),
            pltpu.VMEM_SHARED((ACC_ROWS, EP2), jnp.float32),
            pltpu.SemaphoreType.DMA,
            pltpu.SemaphoreType.DMA,
            pltpu.SemaphoreType.DMA,
            pltpu.SemaphoreType.DMA,
            pltpu.SemaphoreType.DMA,
            pltpu.SemaphoreType.DMA,
        ],
    )
    return edge


def kernel(ent_e, edge_index, edge_type, batch_idx, rel_table, W, b):
    edge_pass = _build_sc_kernels()

    npad = E_PAD - N_EDGES
    src = jnp.concatenate([edge_index[0], jnp.zeros((npad,), jnp.int32)])
    dst = jnp.concatenate(
        [edge_index[1],
         jnp.arange(npad, dtype=jnp.int32) % N_NODES])
    typ = jnp.concatenate(
        [edge_type,
         jnp.full((npad,), NUM_RELS, jnp.int32)])
    sdt = jnp.stack([src.reshape(-1, EB), typ.reshape(-1, EB),
                     dst.reshape(-1, EB)], axis=1)

    tin = jnp.concatenate(
        [ent_e, rel_table,
         jnp.zeros((TROWS - N_NODES - NUM_RELS, EMBED_DIM), jnp.float32)],
        axis=0)
    t1 = _tc_spectrum(tin)
    kmats = _tc_kmats(W.reshape(8, EMBED_DIM, EMBED_DIM))
    bf = b.reshape(8, 1, EMBED_DIM)

    rtabh = t1[N_NODES:N_NODES + RTAB_ROWS]
    acc1 = edge_pass(t1, rtabh, sdt)
    mid = _tc_layer_mid(acc1[:, :TROWS], kmats, bf)

    acc2 = [edge_pass(mid[h], rtabh, sdt) for h in range(NUM_H)]
    acc2s = jnp.stack(acc2, axis=0)

    bidx3 = jnp.concatenate(
        [batch_idx,
         jnp.full((ACC_ROWS - N_NODES,), BATCH, jnp.int32)]
    ).reshape(ACC_ROWS // 128, 1, 128)
    return _tc_layer_out(acc2s, kmats, bf, bidx3)


# speedup vs baseline: 1.2710x; 1.1078x over previous
import functools

import numpy as np
import jax
import jax.numpy as jnp
from jax import lax
from jax.experimental import pallas as pl
from jax.experimental.pallas import tpu as pltpu
from jax.experimental.pallas import tpu_sc as plsc

EMBED_DIM = 128
N_NODES = 10000
N_EDGES = 320000
NUM_RELS = 100
BATCH = 1024
NUM_H = 4

HW = 64
EP2 = 2 * HW
TROWS = 10112
RCOPY = 112
T1ROWS = 13696
ACC_ROWS = 10240
X2_COLS = NUM_H * EMBED_DIM

NC, NS = 2, 16
NW = NC * NS
EB = 64
E_PAD = 327680
EPW = E_PAD // NW
NBLK = EPW // EB
RTAB_ROWS = 112
ZB = 8


def _dft_mats():
    n = EMBED_DIM
    j = np.arange(n)
    ki = np.arange(n // 2 + 1)
    ang = 2.0 * np.pi / n * np.outer(j, ki)
    fc, fs = np.cos(ang), np.sin(ang)
    w = np.where((ki == 0) | (2 * ki == n), 1.0, 2.0)
    iang = 2.0 * np.pi / n * np.outer(ki, j)
    gr = (w[:, None] * np.cos(iang)) / n
    gi = (w[:, None] * np.sin(iang)) / n
    f2 = np.zeros((n, n), np.float32)
    f2[:, :HW] = fc[:, :HW]
    f2[:, HW] = fc[:, HW]
    f2[:, HW + 1:] = -fs[:, 1:HW]
    g2 = np.zeros((n, n), np.float32)
    g2[:HW] = gr[:HW]
    g2[HW] = gr[HW]
    g2[HW + 1:] = -gi[1:HW]
    return f2, g2


_F2P, _G2P = _dft_mats()


def _spectrum_body(x_ref, f_ref, o_ref):
    o_ref[...] = jnp.dot(x_ref[...], f_ref[...],
                         preferred_element_type=jnp.float32)


def _kmat_body(g_ref, w_ref, o_ref):
    o_ref[0] = jnp.dot(g_ref[...], w_ref[0],
                       preferred_element_type=jnp.float32)


def _mid_body(a_ref, k_ref, b_ref, f_ref, o_ref):
    agg = a_ref[0] + a_ref[1]
    z = jnp.maximum(
        jnp.dot(agg, k_ref[0], preferred_element_type=jnp.float32)
        + b_ref[0], 0.0)
    o_ref[0] = jnp.dot(z, f_ref[...], preferred_element_type=jnp.float32)


def _out_body(a_ref, k_ref, b_ref, bidx_ref, o_ref):
    m = pl.program_id(1)
    agg = a_ref[0, 0] + a_ref[0, 1]
    x = jnp.maximum(
        jnp.dot(agg, k_ref[0], preferred_element_type=jnp.float32)
        + b_ref[0], 0.0)
    bb = bidx_ref[0, 0]
    oh = (bb[:, None]
          == lax.broadcasted_iota(jnp.int32, (128, BATCH), 1)
          ).astype(jnp.float32)
    y = jax.lax.dot_general(oh, x, (((0,), (0,)), ((), ())),
                            preferred_element_type=jnp.float32)

    @pl.when(m == 0)
    def _():
        o_ref[...] = jnp.zeros_like(o_ref)
    o_ref[...] += y


def _tc_spectrum(x):
    m = x.shape[0]
    return pl.pallas_call(
        _spectrum_body,
        grid=(m // 128,),
        in_specs=[pl.BlockSpec((128, EMBED_DIM), lambda i: (i, 0)),
                  pl.BlockSpec((EMBED_DIM, EP2), lambda i: (0, 0))],
        out_specs=pl.BlockSpec((128, EP2), lambda i: (i, 0)),
        out_shape=jax.ShapeDtypeStruct((m, EP2), jnp.float32),
    )(x, jnp.asarray(_F2P))


def _tc_kmats(wf):
    return pl.pallas_call(
        _kmat_body,
        grid=(8,),
        in_specs=[pl.BlockSpec((EP2, EMBED_DIM), lambda i: (0, 0)),
                  pl.BlockSpec((1, EMBED_DIM, EMBED_DIM), lambda i: (i, 0, 0))],
        out_specs=pl.BlockSpec((1, EP2, EMBED_DIM), lambda i: (i, 0, 0)),
        out_shape=jax.ShapeDtypeStruct((8, EP2, EMBED_DIM), jnp.float32),
    )(jnp.asarray(_G2P), wf)


def _tc_layer_mid(acc1, kmats, bf):
    return pl.pallas_call(
        _mid_body,
        grid=(NUM_H, TROWS // 128),
        in_specs=[
            pl.BlockSpec((2, 128, EP2), lambda h, m: (0, m, 0)),
            pl.BlockSpec((1, EP2, EMBED_DIM), lambda h, m: (2 * h, 0, 0)),
            pl.BlockSpec((1, 1, EMBED_DIM), lambda h, m: (2 * h, 0, 0)),
            pl.BlockSpec((EMBED_DIM, EP2), lambda h, m: (0, 0)),
        ],
        out_specs=pl.BlockSpec((1, 128, EP2), lambda h, m: (h, m, 0)),
        out_shape=jax.ShapeDtypeStruct((NUM_H, TROWS, EP2), jnp.float32),
    )(acc1, kmats, bf, jnp.asarray(_F2P))


def _tc_layer_out(acc2s, kmats, bf, bidx3):
    return pl.pallas_call(
        _out_body,
        grid=(NUM_H, ACC_ROWS // 128),
        in_specs=[
            pl.BlockSpec((1, 2, 128, EP2), lambda h, m: (h, 0, m, 0)),
            pl.BlockSpec((1, EP2, EMBED_DIM), lambda h, m: (2 * h + 1, 0, 0)),
            pl.BlockSpec((1, 1, EMBED_DIM), lambda h, m: (2 * h + 1, 0, 0)),
            pl.BlockSpec((1, 1, 128), lambda h, m: (m, 0, 0)),
        ],
        out_specs=pl.BlockSpec((BATCH, EMBED_DIM), lambda h, m: (0, h)),
        out_shape=jax.ShapeDtypeStruct((BATCH, X2_COLS), jnp.float32),
    )(acc2s, kmats, bf, bidx3)


def _compute_block(xrows, rrows, prod, m0):
    def pe(e, cc):
        for q in range(HW // 16):
            xa = xrows[e, pl.ds(q * 16, 16)]
            xb = xrows[e, pl.ds(HW + q * 16, 16)]
            ra = rrows[e, pl.ds(q * 16, 16)]
            rb = rrows[e, pl.ds(HW + q * 16, 16)]
            t1 = xa * ra
            t2 = xb * rb
            pre = t1 + t2
            pim = xa * rb - xb * ra
            if q == 0:
                pre = jnp.where(m0, t1, pre)
                pim = jnp.where(m0, t2, pim)
            prod[e, pl.ds(q * 16, 16)] = pre
            prod[e, pl.ds(HW + q * 16, 16)] = pim
        return cc
    lax.fori_loop(0, EB, pe, 0)


def _edge_body(tab, rtab, sdt, out,
               i0, i1, d0, d1, x0, x1, r0, r1, prod, zbuf,
               acc, isa, isb, gsa, gsb, ss):
    c = lax.axis_index("c")
    s = lax.axis_index("s")
    wid = s * NC + c
    z16 = jnp.zeros((16,), jnp.float32)
    m0 = lax.broadcasted_iota(jnp.int32, (16,), 0) == 0

    idx = (i0, i1)
    didx = (d0, d1)
    xb = (x0, x1)
    rb = (r0, r1)
    isx = (isa, isb)
    gs = (gsa, gsb)

    def zb(i, carry):
        for q in range(EP2 // 16):
            zbuf[i, pl.ds(q * 16, 16)] = z16
        return carry
    lax.fori_loop(0, ZB, zb, 0)

    rows_per_sub = ACC_ROWS // NS

    def za(i, carry):
        pltpu.sync_copy(zbuf, acc.at[pl.ds(s * rows_per_sub + i * ZB, ZB)])
        return carry
    lax.fori_loop(0, rows_per_sub // ZB, za, 0)
    plsc.subcore_barrier()

    bbase = wid * NBLK

    def idx_issue(k, u):
        pltpu.async_copy(sdt.at[bbase + k], idx[u], isx[u])

    def idx_wait(u):
        pltpu.make_async_copy(sdt.at[bbase], idx[u], isx[u]).wait()

    def gather_issue(u):
        pltpu.async_copy(tab.at[idx[u].at[0]], xb[u], gs[u])
        pltpu.async_copy(rtab.at[idx[u].at[1]], rb[u], gs[u])

    def gather_wait(u):
        pltpu.make_async_copy(tab.at[idx[u].at[0]], xb[u], gs[u]).wait()
        pltpu.make_async_copy(rtab.at[idx[u].at[1]], rb[u], gs[u]).wait()

    def scat_issue(u):
        pltpu.async_copy(prod, acc.at[didx[u]], ss, add=True)

    def scat_wait():
        pltpu.make_async_copy(prod, acc.at[didx[0]], ss).wait()

    def save_didx(u):
        for j in range(EB // 16):
            didx[u][pl.ds(j * 16, 16)] = idx[u][2, pl.ds(j * 16, 16)]

    def step(k, cur, first):
        nxt = 1 - cur
        gather_wait(cur)
        idx_wait(nxt)
        gather_issue(nxt)
        save_didx(cur)

        @pl.when(k <= NBLK - 3)
        def _():
            idx_issue(k + 2, cur)
        if not first:
            scat_wait()
        _compute_block(xb[cur], rb[cur], prod, m0)
        scat_issue(cur)

    idx_issue(0, 0)
    idx_wait(0)
    gather_issue(0)
    idx_issue(1, 1)
    step(jnp.int32(0), 0, True)
    step(jnp.int32(1), 1, False)

    def pair(i, carry):
        step(2 * i, 0, False)
        step(2 * i + 1, 1, False)
        return carry
    lax.fori_loop(1, NBLK // 2 - 1, pair, 0)
    step(jnp.int32(NBLK - 2), 0, False)
    gather_wait(1)
    save_didx(1)
    scat_wait()
    _compute_block(xb[1], rb[1], prod, m0)
    scat_issue(1)
    scat_wait()
    plsc.subcore_barrier()

    def co(i, carry):
        r0_ = s * rows_per_sub + i * 64
        pltpu.sync_copy(acc.at[pl.ds(r0_, 64)], out.at[c, pl.ds(r0_, 64)])
        return carry
    lax.fori_loop(0, rows_per_sub // 64, co, 0)


@functools.lru_cache(maxsize=None)
def _build_sc_kernels():
    mesh = plsc.VectorSubcoreMesh(core_axis_name="c", subcore_axis_name="s")
    edge = pl.kernel(
        _edge_body, mesh=mesh,
        out_type=jax.ShapeDtypeStruct((NC, ACC_ROWS, EP2), jnp.float32),
        scratch_types=[
            pltpu.VMEM((3, EB), jnp.int32),
            pltpu.VMEM((3, EB), jnp.int32),
            pltpu.VMEM((EB,), jnp.int32),
            pltpu.VMEM((EB,), jnp.int32),
            pltpu.VMEM((EB, EP2), jnp.float32),
            pltpu.VMEM((EB, EP2), jnp.float32),
            pltpu.VMEM((EB, EP2), jnp.float32),
            pltpu.VMEM((EB, EP2), jnp.float32),
            pltpu.VMEM((EB, EP2), jnp.float32),
            pltpu.VMEM((ZB, EP2), jnp.float32),
            pltpu.VMEM_SHARED((ACC_ROWS, EP2), jnp.float32),
            pltpu.SemaphoreType.DMA,
            pltpu.SemaphoreType.DMA,
            pltpu.SemaphoreType.DMA,
            pltpu.SemaphoreType.DMA,
            pltpu.SemaphoreType.DMA,
        ],
    )
    return edge


def kernel(ent_e, edge_index, edge_type, batch_idx, rel_table, W, b):
    edge_pass = _build_sc_kernels()

    npad = E_PAD - N_EDGES
    src = jnp.concatenate([edge_index[0], jnp.zeros((npad,), jnp.int32)])
    dst = jnp.concatenate(
        [edge_index[1],
         jnp.arange(npad, dtype=jnp.int32) % N_NODES])
    typ = jnp.concatenate(
        [edge_type,
         jnp.full((npad,), NUM_RELS, jnp.int32)])
    wk = jnp.arange(E_PAD, dtype=jnp.int32) // EPW
    typr = N_NODES + wk * RCOPY + typ
    sdt = jnp.stack([src.reshape(-1, EB), typr.reshape(-1, EB),
                     dst.reshape(-1, EB)], axis=1)

    relpad = jnp.concatenate(
        [rel_table, jnp.zeros((RCOPY - NUM_RELS, EMBED_DIM), jnp.float32)])
    tin = jnp.concatenate(
        [ent_e, jnp.tile(relpad, (NW, 1)),
         jnp.zeros((T1ROWS - N_NODES - NW * RCOPY, EMBED_DIM), jnp.float32)],
        axis=0)
    t1 = _tc_spectrum(tin)
    kmats = _tc_kmats(W.reshape(8, EMBED_DIM, EMBED_DIM))
    bf = b.reshape(8, 1, EMBED_DIM)

    acc1 = edge_pass(t1, t1, sdt)
    mid = _tc_layer_mid(acc1[:, :TROWS], kmats, bf)

    acc2 = [edge_pass(mid[h], t1, sdt) for h in range(NUM_H)]
    acc2s = jnp.stack(acc2, axis=0)

    bidx3 = jnp.concatenate(
        [batch_idx,
         jnp.full((ACC_ROWS - N_NODES,), BATCH, jnp.int32)]
    ).reshape(ACC_ROWS // 128, 1, 128)
    return _tc_layer_out(acc2s, kmats, bf, bidx3)


# speedup vs baseline: 2.3798x; 1.8724x over previous
import functools

import numpy as np
import jax
import jax.numpy as jnp
from jax import lax
from jax.experimental import pallas as pl
from jax.experimental.pallas import tpu as pltpu
from jax.experimental.pallas import tpu_sc as plsc

EMBED_DIM = 128
N_NODES = 10000
N_EDGES = 320000
NUM_RELS = 100
BATCH = 1024
NUM_H = 4

HW = 64
EP2 = 2 * HW
TROWS = 10112
RCOPY = 112
T1ROWS = 13696
ACC_ROWS = 10240
X2_COLS = NUM_H * EMBED_DIM

NC, NS = 2, 16
NW = NC * NS
EB = 64
E_PAD = 327680
EPW = E_PAD // NW
NBLK = EPW // EB
RTAB_ROWS = 112
ZB = 8


def _dft_mats():
    n = EMBED_DIM
    j = np.arange(n)
    ki = np.arange(n // 2 + 1)
    ang = 2.0 * np.pi / n * np.outer(j, ki)
    fc, fs = np.cos(ang), np.sin(ang)
    w = np.where((ki == 0) | (2 * ki == n), 1.0, 2.0)
    iang = 2.0 * np.pi / n * np.outer(ki, j)
    gr = (w[:, None] * np.cos(iang)) / n
    gi = (w[:, None] * np.sin(iang)) / n
    f2 = np.zeros((n, n), np.float32)
    f2[:, :HW] = fc[:, :HW]
    f2[:, HW] = fc[:, HW]
    f2[:, HW + 1:] = -fs[:, 1:HW]
    g2 = np.zeros((n, n), np.float32)
    g2[:HW] = gr[:HW]
    g2[HW] = gr[HW]
    g2[HW + 1:] = -gi[1:HW]
    return f2, g2


_F2P, _G2P = _dft_mats()


def _spectrum_body(x_ref, f_ref, o_ref):
    o_ref[...] = jnp.dot(x_ref[...], f_ref[...],
                         preferred_element_type=jnp.float32)


def _kmat_body(g_ref, w_ref, o_ref):
    o_ref[0] = jnp.dot(g_ref[...], w_ref[0],
                       preferred_element_type=jnp.float32)


def _mid_body(a_ref, k_ref, b_ref, f_ref, o_ref):
    agg = a_ref[0] + a_ref[1]
    z = jnp.maximum(
        jnp.dot(agg, k_ref[0], preferred_element_type=jnp.float32)
        + b_ref[0], 0.0)
    o_ref[0] = jnp.dot(z, f_ref[...], preferred_element_type=jnp.float32)


def _out_body(a_ref, k_ref, b_ref, bidx_ref, o_ref):
    m = pl.program_id(1)
    agg = a_ref[0, 0] + a_ref[0, 1]
    x = jnp.maximum(
        jnp.dot(agg, k_ref[0], preferred_element_type=jnp.float32)
        + b_ref[0], 0.0)
    bb = bidx_ref[0, 0]
    oh = (bb[:, None]
          == lax.broadcasted_iota(jnp.int32, (128, BATCH), 1)
          ).astype(jnp.float32)
    y = jax.lax.dot_general(oh, x, (((0,), (0,)), ((), ())),
                            preferred_element_type=jnp.float32)

    @pl.when(m == 0)
    def _():
        o_ref[...] = jnp.zeros_like(o_ref)
    o_ref[...] += y


def _tc_spectrum(x):
    m = x.shape[0]
    return pl.pallas_call(
        _spectrum_body,
        grid=(m // 128,),
        in_specs=[pl.BlockSpec((128, EMBED_DIM), lambda i: (i, 0)),
                  pl.BlockSpec((EMBED_DIM, EP2), lambda i: (0, 0))],
        out_specs=pl.BlockSpec((128, EP2), lambda i: (i, 0)),
        out_shape=jax.ShapeDtypeStruct((m, EP2), jnp.float32),
    )(x, jnp.asarray(_F2P))


def _tc_kmats(wf):
    return pl.pallas_call(
        _kmat_body,
        grid=(8,),
        in_specs=[pl.BlockSpec((EP2, EMBED_DIM), lambda i: (0, 0)),
                  pl.BlockSpec((1, EMBED_DIM, EMBED_DIM), lambda i: (i, 0, 0))],
        out_specs=pl.BlockSpec((1, EP2, EMBED_DIM), lambda i: (i, 0, 0)),
        out_shape=jax.ShapeDtypeStruct((8, EP2, EMBED_DIM), jnp.float32),
    )(jnp.asarray(_G2P), wf)


def _tc_layer_mid(acc1, kmats, bf):
    return pl.pallas_call(
        _mid_body,
        grid=(NUM_H, TROWS // 128),
        in_specs=[
            pl.BlockSpec((2, 128, EP2), lambda h, m: (0, m, 0)),
            pl.BlockSpec((1, EP2, EMBED_DIM), lambda h, m: (2 * h, 0, 0)),
            pl.BlockSpec((1, 1, EMBED_DIM), lambda h, m: (2 * h, 0, 0)),
            pl.BlockSpec((EMBED_DIM, EP2), lambda h, m: (0, 0)),
        ],
        out_specs=pl.BlockSpec((1, 128, EP2), lambda h, m: (h, m, 0)),
        out_shape=jax.ShapeDtypeStruct((NUM_H, TROWS, EP2), jnp.float32),
    )(acc1, kmats, bf, jnp.asarray(_F2P))


def _tc_layer_out(acc2s, kmats, bf, bidx3):
    return pl.pallas_call(
        _out_body,
        grid=(NUM_H, ACC_ROWS // 128),
        in_specs=[
            pl.BlockSpec((1, 2, 128, EP2), lambda h, m: (h, 0, m, 0)),
            pl.BlockSpec((1, EP2, EMBED_DIM), lambda h, m: (2 * h + 1, 0, 0)),
            pl.BlockSpec((1, 1, EMBED_DIM), lambda h, m: (2 * h + 1, 0, 0)),
            pl.BlockSpec((1, 1, 128), lambda h, m: (m, 0, 0)),
        ],
        out_specs=pl.BlockSpec((BATCH, EMBED_DIM), lambda h, m: (0, h)),
        out_shape=jax.ShapeDtypeStruct((BATCH, X2_COLS), jnp.float32),
    )(acc2s, kmats, bf, bidx3)


def _compute_block(xrows, rrows, prod, m0):
    def pe(e, cc):
        for q in range(HW // 16):
            xa = xrows[e, pl.ds(q * 16, 16)]
            xb = xrows[e, pl.ds(HW + q * 16, 16)]
            ra = rrows[e, pl.ds(q * 16, 16)]
            rb = rrows[e, pl.ds(HW + q * 16, 16)]
            t1 = xa * ra
            t2 = xb * rb
            pre = t1 + t2
            pim = xa * rb - xb * ra
            if q == 0:
                pre = jnp.where(m0, t1, pre)
                pim = jnp.where(m0, t2, pim)
            prod[e, pl.ds(q * 16, 16)] = pre
            prod[e, pl.ds(HW + q * 16, 16)] = pim
        return cc
    lax.fori_loop(0, EB, pe, 0)


def _edge_body(tab, rtab, sdt, out,
               i0, i1, d0, d1, x0, x1, r0, r1, prod, zbuf,
               acc, isa, isb, gsa, gsb, ss):
    c = lax.axis_index("c")
    s = lax.axis_index("s")
    wid = s * NC + c
    z16 = jnp.zeros((16,), jnp.float32)
    m0 = lax.broadcasted_iota(jnp.int32, (16,), 0) == 0

    idx = (i0, i1)
    didx = (d0, d1)
    xb = (x0, x1)
    rb = (r0, r1)
    isx = (isa, isb)
    gs = (gsa, gsb)

    def zb(i, carry):
        for q in range(EP2 // 16):
            zbuf[i, pl.ds(q * 16, 16)] = z16
        return carry
    lax.fori_loop(0, ZB, zb, 0)

    rows_per_sub = ACC_ROWS // NS

    def za(i, carry):
        pltpu.sync_copy(zbuf, acc.at[pl.ds(s * rows_per_sub + i * ZB, ZB)])
        return carry
    lax.fori_loop(0, rows_per_sub // ZB, za, 0)
    plsc.subcore_barrier()

    bbase = wid * NBLK

    def idx_issue(k, u):
        pltpu.async_copy(sdt.at[bbase + k], idx[u], isx[u])

    def idx_wait(u):
        pltpu.make_async_copy(sdt.at[bbase], idx[u], isx[u]).wait()

    def gather_issue(u):
        pltpu.async_copy(tab.at[idx[u].at[0]], xb[u], gs[u])
        pltpu.async_copy(rtab.at[idx[u].at[1]], rb[u], gs[u])

    def gather_wait(u):
        pltpu.make_async_copy(tab.at[idx[u].at[0]], xb[u], gs[u]).wait()
        pltpu.make_async_copy(rtab.at[idx[u].at[1]], rb[u], gs[u]).wait()

    def scat_issue(u):
        pltpu.async_copy(prod, acc.at[didx[u]], ss, add=True)

    def scat_wait():
        pltpu.make_async_copy(prod, acc.at[didx[0]], ss).wait()

    def save_didx(u):
        for j in range(EB // 16):
            didx[u][pl.ds(j * 16, 16)] = idx[u][2, pl.ds(j * 16, 16)]

    def step(k, cur, first):
        nxt = 1 - cur
        gather_wait(cur)
        idx_wait(nxt)
        gather_issue(nxt)
        save_didx(cur)

        @pl.when(k <= NBLK - 3)
        def _():
            idx_issue(k + 2, cur)
        if not first:
            scat_wait()
        _compute_block(xb[cur], rb[cur], prod, m0)
        scat_issue(cur)

    idx_issue(0, 0)
    idx_wait(0)
    gather_issue(0)
    idx_issue(1, 1)
    step(jnp.int32(0), 0, True)
    step(jnp.int32(1), 1, False)

    def pair(i, carry):
        step(2 * i, 0, False)
        step(2 * i + 1, 1, False)
        return carry
    lax.fori_loop(1, NBLK // 2 - 1, pair, 0)
    step(jnp.int32(NBLK - 2), 0, False)
    gather_wait(1)
    save_didx(1)
    scat_wait()
    _compute_block(xb[1], rb[1], prod, m0)
    scat_issue(1)
    scat_wait()
    plsc.subcore_barrier()

    def co(i, carry):
        r0_ = s * rows_per_sub + i * 64
        pltpu.sync_copy(acc.at[pl.ds(r0_, 64)], out.at[c, pl.ds(r0_, 64)])
        return carry
    lax.fori_loop(0, rows_per_sub // 64, co, 0)


@functools.lru_cache(maxsize=None)
def _build_sc_kernels():
    mesh = plsc.VectorSubcoreMesh(core_axis_name="c", subcore_axis_name="s")
    edge = pl.kernel(
        _edge_body, mesh=mesh,
        out_type=jax.ShapeDtypeStruct((NC, ACC_ROWS, EP2), jnp.float32),
        scratch_types=[
            pltpu.VMEM((3, EB), jnp.int32),
            pltpu.VMEM((3, EB), jnp.int32),
            pltpu.VMEM((EB,), jnp.int32),
            pltpu.VMEM((EB,), jnp.int32),
            pltpu.VMEM((EB, EP2), jnp.float32),
            pltpu.VMEM((EB, EP2), jnp.float32),
            pltpu.VMEM((EB, EP2), jnp.float32),
            pltpu.VMEM((EB, EP2), jnp.float32),
            pltpu.VMEM((EB, EP2), jnp.float32),
            pltpu.VMEM((ZB, EP2), jnp.float32),
            pltpu.VMEM_SHARED((ACC_ROWS, EP2), jnp.float32),
            pltpu.SemaphoreType.DMA,
            pltpu.SemaphoreType.DMA,
            pltpu.SemaphoreType.DMA,
            pltpu.SemaphoreType.DMA,
            pltpu.SemaphoreType.DMA,
        ],
    )
    return edge


def kernel(ent_e, edge_index, edge_type, batch_idx, rel_table, W, b):
    edge_pass = _build_sc_kernels()

    npad = E_PAD - N_EDGES
    pad_i = jnp.arange(npad, dtype=jnp.int32)
    src = jnp.concatenate([edge_index[0], pad_i % N_NODES])
    dst = jnp.concatenate([edge_index[1], pad_i % N_NODES])
    wk = jnp.arange(N_EDGES, dtype=jnp.int32) // EPW
    typr = jnp.concatenate(
        [N_NODES + wk * RCOPY + edge_type,
         N_NODES + (pad_i % NW) * RCOPY + NUM_RELS
         + (pad_i // NW) % (RCOPY - NUM_RELS)])
    sdt = jnp.stack([src.reshape(-1, EB), typr.reshape(-1, EB),
                     dst.reshape(-1, EB)], axis=1)

    relpad = jnp.concatenate(
        [rel_table, jnp.zeros((RCOPY - NUM_RELS, EMBED_DIM), jnp.float32)])
    tin = jnp.concatenate(
        [ent_e, jnp.tile(relpad, (NW, 1)),
         jnp.zeros((T1ROWS - N_NODES - NW * RCOPY, EMBED_DIM), jnp.float32)],
        axis=0)
    t1 = _tc_spectrum(tin)
    kmats = _tc_kmats(W.reshape(8, EMBED_DIM, EMBED_DIM))
    bf = b.reshape(8, 1, EMBED_DIM)

    acc1 = edge_pass(t1, t1, sdt)
    mid = _tc_layer_mid(acc1[:, :TROWS], kmats, bf)

    acc2 = [edge_pass(mid[h], t1, sdt) for h in range(NUM_H)]
    acc2s = jnp.stack(acc2, axis=0)

    bidx3 = jnp.concatenate(
        [batch_idx,
         jnp.full((ACC_ROWS - N_NODES,), BATCH, jnp.int32)]
    ).reshape(ACC_ROWS // 128, 1, 128)
    return _tc_layer_out(acc2s, kmats, bf, bidx3)


# speedup vs baseline: 2.4568x; 1.0324x over previous
import functools

import numpy as np
import jax
import jax.numpy as jnp
from jax import lax
from jax.experimental import pallas as pl
from jax.experimental.pallas import tpu as pltpu
from jax.experimental.pallas import tpu_sc as plsc

EMBED_DIM = 128
N_NODES = 10000
N_EDGES = 320000
NUM_RELS = 100
BATCH = 1024
NUM_H = 4

HW = 64
EP2 = 2 * HW
TROWS = 10112
RCOPY = 112
T1ROWS = 13696
ACC_ROWS = 10240
X2_COLS = NUM_H * EMBED_DIM

NC, NS = 2, 16
NW = NC * NS
EB = 64
E_PAD = 327680
EPW = E_PAD // NW
NBLK = EPW // EB
RTAB_ROWS = 112
ZB = 8


def _dft_mats():
    n = EMBED_DIM
    j = np.arange(n)
    ki = np.arange(n // 2 + 1)
    ang = 2.0 * np.pi / n * np.outer(j, ki)
    fc, fs = np.cos(ang), np.sin(ang)
    w = np.where((ki == 0) | (2 * ki == n), 1.0, 2.0)
    iang = 2.0 * np.pi / n * np.outer(ki, j)
    gr = (w[:, None] * np.cos(iang)) / n
    gi = (w[:, None] * np.sin(iang)) / n
    f2 = np.zeros((n, n), np.float32)
    f2[:, :HW] = fc[:, :HW]
    f2[:, HW] = fc[:, HW]
    f2[:, HW + 1:] = -fs[:, 1:HW]
    g2 = np.zeros((n, n), np.float32)
    g2[:HW] = gr[:HW]
    g2[HW] = gr[HW]
    g2[HW + 1:] = -gi[1:HW]
    return f2, g2


_F2P, _G2P = _dft_mats()


def _spectrum_body(x_ref, f_ref, o_ref):
    o_ref[...] = jnp.dot(x_ref[...], f_ref[...],
                         preferred_element_type=jnp.float32)


def _kmat_body(g_ref, w_ref, o_ref):
    o_ref[0] = jnp.dot(g_ref[...], w_ref[0],
                       preferred_element_type=jnp.float32)


def _mid_body(a_ref, k_ref, b_ref, f_ref, o_ref):
    agg = a_ref[0] + a_ref[1]
    z = jnp.maximum(
        jnp.dot(agg, k_ref[0], preferred_element_type=jnp.float32)
        + b_ref[0], 0.0)
    o_ref[0] = jnp.dot(z, f_ref[...], preferred_element_type=jnp.float32)


def _out_body(a_ref, k_ref, b_ref, bidx_ref, o_ref):
    m = pl.program_id(1)
    agg = a_ref[0, 0] + a_ref[0, 1]
    x = jnp.maximum(
        jnp.dot(agg, k_ref[0], preferred_element_type=jnp.float32)
        + b_ref[0], 0.0)
    bb = bidx_ref[0, 0]
    oh = (bb[:, None]
          == lax.broadcasted_iota(jnp.int32, (128, BATCH), 1)
          ).astype(jnp.float32)
    y = jax.lax.dot_general(oh, x, (((0,), (0,)), ((), ())),
                            preferred_element_type=jnp.float32)

    @pl.when(m == 0)
    def _():
        o_ref[...] = jnp.zeros_like(o_ref)
    o_ref[...] += y


def _tc_spectrum(x):
    m = x.shape[0]
    return pl.pallas_call(
        _spectrum_body,
        grid=(m // 128,),
        in_specs=[pl.BlockSpec((128, EMBED_DIM), lambda i: (i, 0)),
                  pl.BlockSpec((EMBED_DIM, EP2), lambda i: (0, 0))],
        out_specs=pl.BlockSpec((128, EP2), lambda i: (i, 0)),
        out_shape=jax.ShapeDtypeStruct((m, EP2), jnp.float32),
    )(x, jnp.asarray(_F2P))


def _tc_kmats(wf):
    return pl.pallas_call(
        _kmat_body,
        grid=(8,),
        in_specs=[pl.BlockSpec((EP2, EMBED_DIM), lambda i: (0, 0)),
                  pl.BlockSpec((1, EMBED_DIM, EMBED_DIM), lambda i: (i, 0, 0))],
        out_specs=pl.BlockSpec((1, EP2, EMBED_DIM), lambda i: (i, 0, 0)),
        out_shape=jax.ShapeDtypeStruct((8, EP2, EMBED_DIM), jnp.float32),
    )(jnp.asarray(_G2P), wf)


def _tc_layer_mid(acc1, kmats, bf):
    return pl.pallas_call(
        _mid_body,
        grid=(NUM_H, TROWS // 128),
        in_specs=[
            pl.BlockSpec((2, 128, EP2), lambda h, m: (0, m, 0)),
            pl.BlockSpec((1, EP2, EMBED_DIM), lambda h, m: (2 * h, 0, 0)),
            pl.BlockSpec((1, 1, EMBED_DIM), lambda h, m: (2 * h, 0, 0)),
            pl.BlockSpec((EMBED_DIM, EP2), lambda h, m: (0, 0)),
        ],
        out_specs=pl.BlockSpec((1, 128, EP2), lambda h, m: (h, m, 0)),
        out_shape=jax.ShapeDtypeStruct((NUM_H, TROWS, EP2), jnp.float32),
    )(acc1, kmats, bf, jnp.asarray(_F2P))


def _tc_layer_out(acc2s, kmats, bf, bidx3):
    return pl.pallas_call(
        _out_body,
        grid=(NUM_H, ACC_ROWS // 128),
        in_specs=[
            pl.BlockSpec((1, 2, 128, EP2), lambda h, m: (h, 0, m, 0)),
            pl.BlockSpec((1, EP2, EMBED_DIM), lambda h, m: (2 * h + 1, 0, 0)),
            pl.BlockSpec((1, 1, EMBED_DIM), lambda h, m: (2 * h + 1, 0, 0)),
            pl.BlockSpec((1, 1, 128), lambda h, m: (m, 0, 0)),
        ],
        out_specs=pl.BlockSpec((BATCH, EMBED_DIM), lambda h, m: (0, h)),
        out_shape=jax.ShapeDtypeStruct((BATCH, X2_COLS), jnp.float32),
    )(acc2s, kmats, bf, bidx3)


def _compute_block(xrows, rrows, prod, m0):
    def pe(e, cc):
        for q in range(HW // 16):
            xa = xrows[e, pl.ds(q * 16, 16)]
            xb = xrows[e, pl.ds(HW + q * 16, 16)]
            ra = rrows[e, pl.ds(q * 16, 16)]
            rb = rrows[e, pl.ds(HW + q * 16, 16)]
            t1 = xa * ra
            t2 = xb * rb
            pre = t1 + t2
            pim = xa * rb - xb * ra
            if q == 0:
                pre = jnp.where(m0, t1, pre)
                pim = jnp.where(m0, t2, pim)
            prod[e, pl.ds(q * 16, 16)] = pre
            prod[e, pl.ds(HW + q * 16, 16)] = pim
        return cc
    lax.fori_loop(0, EB, pe, 0)


def _run_pipeline(tab_h, rtab, sdt, acc, idx, didx, xb, rb, prod, isx, gs,
                  ss, bbase, m0):
    def idx_issue(k, u):
        pltpu.async_copy(sdt.at[bbase + k], idx[u], isx[u])

    def idx_wait(u):
        pltpu.make_async_copy(sdt.at[bbase], idx[u], isx[u]).wait()

    def gather_issue(u):
        pltpu.async_copy(tab_h.at[idx[u].at[0]], xb[u], gs[u])
        pltpu.async_copy(rtab.at[idx[u].at[1]], rb[u], gs[u])

    def gather_wait(u):
        pltpu.make_async_copy(tab_h.at[idx[u].at[0]], xb[u], gs[u]).wait()
        pltpu.make_async_copy(rtab.at[idx[u].at[1]], rb[u], gs[u]).wait()

    def scat_issue(u):
        pltpu.async_copy(prod, acc.at[didx[u]], ss, add=True)

    def scat_wait():
        pltpu.make_async_copy(prod, acc.at[didx[0]], ss).wait()

    def save_didx(u):
        for j in range(EB // 16):
            didx[u][pl.ds(j * 16, 16)] = idx[u][2, pl.ds(j * 16, 16)]

    def step(k, cur, first):
        nxt = 1 - cur
        gather_wait(cur)
        idx_wait(nxt)
        gather_issue(nxt)
        save_didx(cur)

        @pl.when(k <= NBLK - 3)
        def _():
            idx_issue(k + 2, cur)
        if not first:
            scat_wait()
        _compute_block(xb[cur], rb[cur], prod, m0)
        scat_issue(cur)

    idx_issue(0, 0)
    idx_wait(0)
    gather_issue(0)
    idx_issue(1, 1)
    step(jnp.int32(0), 0, True)
    step(jnp.int32(1), 1, False)

    def pair(i, carry):
        step(2 * i, 0, False)
        step(2 * i + 1, 1, False)
        return carry
    lax.fori_loop(1, NBLK // 2 - 1, pair, 0)
    step(jnp.int32(NBLK - 2), 0, False)
    gather_wait(1)
    save_didx(1)
    scat_wait()
    _compute_block(xb[1], rb[1], prod, m0)
    scat_issue(1)
    scat_wait()


def _edge_body(tab, rtab, sdt, out,
               i0, i1, d0, d1, x0, x1, r0, r1, prod, zbuf,
               acc, isa, isb, gsa, gsb, ss):
    c = lax.axis_index("c")
    s = lax.axis_index("s")
    wid = s * NC + c
    z16 = jnp.zeros((16,), jnp.float32)
    m0 = lax.broadcasted_iota(jnp.int32, (16,), 0) == 0
    rows_per_sub = ACC_ROWS // NS

    def zb(i, carry):
        for q in range(EP2 // 16):
            zbuf[i, pl.ds(q * 16, 16)] = z16
        return carry
    lax.fori_loop(0, ZB, zb, 0)

    def za(i, carry):
        pltpu.sync_copy(zbuf, acc.at[pl.ds(s * rows_per_sub + i * ZB, ZB)])
        return carry
    lax.fori_loop(0, rows_per_sub // ZB, za, 0)
    plsc.subcore_barrier()

    _run_pipeline(tab, rtab, sdt, acc, (i0, i1), (d0, d1), (x0, x1),
                  (r0, r1), prod, (isa, isb), (gsa, gsb), ss,
                  wid * NBLK, m0)
    plsc.subcore_barrier()

    def co(i, carry):
        r0_ = s * rows_per_sub + i * 64
        pltpu.sync_copy(acc.at[pl.ds(r0_, 64)], out.at[c, pl.ds(r0_, 64)])
        return carry
    lax.fori_loop(0, rows_per_sub // 64, co, 0)


def _edge4_body(tabs, rtab, sdt, out,
                i0, i1, d0, d1, x0, x1, r0, r1, prod, zbuf,
                acc, isa, isb, gsa, gsb, ss):
    c = lax.axis_index("c")
    s = lax.axis_index("s")
    wid = s * NC + c
    z16 = jnp.zeros((16,), jnp.float32)
    m0 = lax.broadcasted_iota(jnp.int32, (16,), 0) == 0
    rows_per_sub = ACC_ROWS // NS

    def zb(i, carry):
        for q in range(EP2 // 16):
            zbuf[i, pl.ds(q * 16, 16)] = z16
        return carry
    lax.fori_loop(0, ZB, zb, 0)

    def zero_acc():
        def za(i, carry):
            pltpu.sync_copy(zbuf,
                            acc.at[pl.ds(s * rows_per_sub + i * ZB, ZB)])
            return carry
        lax.fori_loop(0, rows_per_sub // ZB, za, 0)

    zero_acc()
    plsc.subcore_barrier()

    for h in range(NUM_H):
        _run_pipeline(tabs.at[h], rtab, sdt, acc, (i0, i1), (d0, d1),
                      (x0, x1), (r0, r1), prod, (isa, isb), (gsa, gsb), ss,
                      wid * NBLK, m0)
        plsc.subcore_barrier()

        def co(i, carry):
            r0_ = s * rows_per_sub + i * 64
            pltpu.sync_copy(acc.at[pl.ds(r0_, 64)],
                            out.at[h, c, pl.ds(r0_, 64)])
            return carry
        lax.fori_loop(0, rows_per_sub // 64, co, 0)
        if h < NUM_H - 1:
            zero_acc()
            plsc.subcore_barrier()


@functools.lru_cache(maxsize=None)
def _build_sc_kernels():
    mesh = plsc.VectorSubcoreMesh(core_axis_name="c", subcore_axis_name="s")
    edge = pl.kernel(
        _edge_body, mesh=mesh,
        out_type=jax.ShapeDtypeStruct((NC, ACC_ROWS, EP2), jnp.float32),
        scratch_types=[
            pltpu.VMEM((3, EB), jnp.int32),
            pltpu.VMEM((3, EB), jnp.int32),
            pltpu.VMEM((EB,), jnp.int32),
            pltpu.VMEM((EB,), jnp.int32),
            pltpu.VMEM((EB, EP2), jnp.float32),
            pltpu.VMEM((EB, EP2), jnp.float32),
            pltpu.VMEM((EB, EP2), jnp.float32),
            pltpu.VMEM((EB, EP2), jnp.float32),
            pltpu.VMEM((EB, EP2), jnp.float32),
            pltpu.VMEM((ZB, EP2), jnp.float32),
            pltpu.VMEM_SHARED((ACC_ROWS, EP2), jnp.float32),
            pltpu.SemaphoreType.DMA,
            pltpu.SemaphoreType.DMA,
            pltpu.SemaphoreType.DMA,
            pltpu.SemaphoreType.DMA,
            pltpu.SemaphoreType.DMA,
        ],
    )
    scratches = [
        pltpu.VMEM((3, EB), jnp.int32),
        pltpu.VMEM((3, EB), jnp.int32),
        pltpu.VMEM((EB,), jnp.int32),
        pltpu.VMEM((EB,), jnp.int32),
        pltpu.VMEM((EB, EP2), jnp.float32),
        pltpu.VMEM((EB, EP2), jnp.float32),
        pltpu.VMEM((EB, EP2), jnp.float32),
        pltpu.VMEM((EB, EP2), jnp.float32),
        pltpu.VMEM((EB, EP2), jnp.float32),
        pltpu.VMEM((ZB, EP2), jnp.float32),
        pltpu.VMEM_SHARED((ACC_ROWS, EP2), jnp.float32),
        pltpu.SemaphoreType.DMA,
        pltpu.SemaphoreType.DMA,
        pltpu.SemaphoreType.DMA,
        pltpu.SemaphoreType.DMA,
        pltpu.SemaphoreType.DMA,
    ]
    edge4 = pl.kernel(
        _edge4_body, mesh=mesh,
        out_type=jax.ShapeDtypeStruct((NUM_H, NC, ACC_ROWS, EP2),
                                      jnp.float32),
        scratch_types=scratches,
    )
    return edge, edge4


def kernel(ent_e, edge_index, edge_type, batch_idx, rel_table, W, b):
    edge_pass, edge4_pass = _build_sc_kernels()

    npad = E_PAD - N_EDGES
    pad_i = jnp.arange(npad, dtype=jnp.int32)
    src = jnp.concatenate([edge_index[0], pad_i % N_NODES])
    dst = jnp.concatenate([edge_index[1], pad_i % N_NODES])
    wk = jnp.arange(N_EDGES, dtype=jnp.int32) // EPW
    typr = jnp.concatenate(
        [N_NODES + wk * RCOPY + edge_type,
         N_NODES + (pad_i % NW) * RCOPY + NUM_RELS
         + (pad_i // NW) % (RCOPY - NUM_RELS)])
    sdt = jnp.stack([src.reshape(-1, EB), typr.reshape(-1, EB),
                     dst.reshape(-1, EB)], axis=1)

    relpad = jnp.concatenate(
        [rel_table, jnp.zeros((RCOPY - NUM_RELS, EMBED_DIM), jnp.float32)])
    tin = jnp.concatenate(
        [ent_e, jnp.tile(relpad, (NW, 1)),
         jnp.zeros((T1ROWS - N_NODES - NW * RCOPY, EMBED_DIM), jnp.float32)],
        axis=0)
    t1 = _tc_spectrum(tin)
    kmats = _tc_kmats(W.reshape(8, EMBED_DIM, EMBED_DIM))
    bf = b.reshape(8, 1, EMBED_DIM)

    acc1 = edge_pass(t1, t1, sdt)
    mid = _tc_layer_mid(acc1[:, :TROWS], kmats, bf)

    acc2s = edge4_pass(mid, t1, sdt)

    bidx3 = jnp.concatenate(
        [batch_idx,
         jnp.full((ACC_ROWS - N_NODES,), BATCH, jnp.int32)]
    ).reshape(ACC_ROWS // 128, 1, 128)
    return _tc_layer_out(acc2s, kmats, bf, bidx3)
